# baseline (speedup 1.0000x reference)
import functools

import jax
import jax.numpy as jnp
from jax import lax
from jax.experimental import pallas as pl
from jax.experimental.pallas import tpu as pltpu
from jax.experimental.pallas import tpu_sc as plsc

BATCH = 16384
F = 512
LANES = 16
NC = 2
NS = 16
NW = NC * NS
ROWS_PER_W = BATCH // NW
R = 32
NCHUNK = ROWS_PER_W // R
NPAIR = NCHUNK // 2
NJ = F // LANES

_mesh = plsc.VectorSubcoreMesh(core_axis_name="c", subcore_axis_name="s")


@functools.partial(
    pl.kernel,
    out_type=jax.ShapeDtypeStruct((BATCH, F), jnp.float32),
    mesh=_mesh,
    compiler_params=pltpu.CompilerParams(needs_layout_passes=False),
    scratch_types=[
        pltpu.VMEM((F,), jnp.int32),
        pltpu.VMEM((R, F), jnp.float32),
        pltpu.VMEM((R, F), jnp.float32),
        pltpu.VMEM((R, F), jnp.float32),
        pltpu.VMEM((R, F), jnp.float32),
        pltpu.SemaphoreType.DMA,
        pltpu.SemaphoreType.DMA,
        pltpu.SemaphoreType.DMA,
        pltpu.SemaphoreType.DMA,
    ],
)
def _sc_gather(x_hbm, idx_hbm, out_hbm, idx_v, xb0, xb1, ob0, ob1,
               si0, si1, so0, so1):
    wid = lax.axis_index("s") * NC + lax.axis_index("c")
    base = wid * ROWS_PER_W

    pltpu.sync_copy(idx_hbm, idx_v)
    cols = [idx_v[pl.ds(j * LANES, LANES)] for j in range(NJ)]

    def start_in(ci, buf, sem):
        pltpu.async_copy(x_hbm.at[pl.ds(base + ci * R, R)], buf, sem)

    def start_out(ci, buf, sem):
        pltpu.async_copy(buf, out_hbm.at[pl.ds(base + ci * R, R)], sem)

    def wait_in(buf, sem):
        pltpu.make_async_copy(x_hbm.at[pl.ds(base, R)], buf, sem).wait()

    def wait_out(buf, sem):
        pltpu.make_async_copy(buf, out_hbm.at[pl.ds(base, R)], sem).wait()

    def gather(xb, ob):
        for j in range(NJ):
            col = cols[j]

            @plsc.parallel_loop(0, R, unroll=8)
            def _row(r):
                row = jnp.full((LANES,), r, dtype=jnp.int32)
                vals = plsc.load_gather(xb, [row, col])
                ob[r, pl.ds(j * LANES, LANES)] = vals

    start_in(0, xb0, si0)
    start_in(1, xb1, si1)

    def pair_body(g, carry):
        for b, (xb, ob, si, so) in enumerate(
            ((xb0, ob0, si0, so0), (xb1, ob1, si1, so1))):
            ci = 2 * g + b
            wait_in(xb, si)

            @pl.when(g > 0)
            def _():
                wait_out(ob, so)

            gather(xb, ob)
            start_out(ci, ob, so)

            @pl.when(g < NPAIR - 1)
            def _():
                start_in(ci + 2, xb, si)
        return carry

    lax.fori_loop(0, NPAIR, pair_body, 0)

    wait_out(ob0, so0)
    wait_out(ob1, so1)


def kernel(x, sparse_index):
    return _sc_gather(x, sparse_index.astype(jnp.int32))


# speedup vs baseline: 1.3274x; 1.3274x over previous
import functools

import jax
import jax.numpy as jnp
from jax import lax
from jax.experimental import pallas as pl
from jax.experimental.pallas import tpu as pltpu
from jax.experimental.pallas import tpu_sc as plsc

BATCH = 16384
F = 512
LANES = 16
NC = 2
NS = 16
NW = NC * NS
ROWS_PER_W = BATCH // NW
R = 32
NCHUNK = ROWS_PER_W // R
NPAIR = NCHUNK // 2
NJ = F // LANES

_mesh = plsc.VectorSubcoreMesh(core_axis_name="c", subcore_axis_name="s")


@functools.partial(
    pl.kernel,
    out_type=jax.ShapeDtypeStruct((BATCH, F), jnp.float32),
    mesh=_mesh,
    compiler_params=pltpu.CompilerParams(needs_layout_passes=False),
    scratch_types=[
        pltpu.VMEM((F,), jnp.int32),
        pltpu.VMEM((R, F), jnp.float32),
        pltpu.VMEM((R, F), jnp.float32),
        pltpu.VMEM((R, F), jnp.float32),
        pltpu.VMEM((R, F), jnp.float32),
        pltpu.SemaphoreType.DMA,
        pltpu.SemaphoreType.DMA,
        pltpu.SemaphoreType.DMA,
        pltpu.SemaphoreType.DMA,
    ],
)
def _sc_gather(x_hbm, idx_hbm, out_hbm, idx_v, xb0, xb1, ob0, ob1,
               si0, si1, so0, so1):
    wid = lax.axis_index("s") * NC + lax.axis_index("c")
    base = wid * ROWS_PER_W

    pltpu.sync_copy(idx_hbm, idx_v)
    cols = [idx_v[pl.ds(j * LANES, LANES)] for j in range(NJ)]

    def start_in(ci, buf, sem):
        pltpu.async_copy(x_hbm.at[pl.ds(base + ci * R, R)], buf, sem)

    def start_out(ci, buf, sem):
        pltpu.async_copy(buf, out_hbm.at[pl.ds(base + ci * R, R)], sem)

    def wait_in(buf, sem):
        pltpu.make_async_copy(x_hbm.at[pl.ds(base, R)], buf, sem).wait()

    def wait_out(buf, sem):
        pltpu.make_async_copy(buf, out_hbm.at[pl.ds(base, R)], sem).wait()

    def gather(xb, ob):
        @plsc.parallel_loop(0, R, unroll=2)
        def _row(r):
            row = jnp.full((LANES,), r, dtype=jnp.int32)
            for j in range(NJ):
                vals = plsc.load_gather(xb, [row, cols[j]])
                ob[r, pl.ds(j * LANES, LANES)] = vals

    start_in(0, xb0, si0)
    start_in(1, xb1, si1)

    def pair_body(g, carry):
        for b, (xb, ob, si, so) in enumerate(
            ((xb0, ob0, si0, so0), (xb1, ob1, si1, so1))):
            ci = 2 * g + b
            wait_in(xb, si)

            @pl.when(g > 0)
            def _():
                wait_out(ob, so)

            gather(xb, ob)
            start_out(ci, ob, so)

            @pl.when(g < NPAIR - 1)
            def _():
                start_in(ci + 2, xb, si)
        return carry

    lax.fori_loop(0, NPAIR, pair_body, 0)

    wait_out(ob0, so0)
    wait_out(ob1, so1)


def kernel(x, sparse_index):
    return _sc_gather(x, sparse_index.astype(jnp.int32))


# speedup vs baseline: 1.4512x; 1.0932x over previous
import functools

import jax
import jax.numpy as jnp
from jax import lax
from jax.experimental import pallas as pl
from jax.experimental.pallas import tpu as pltpu
from jax.experimental.pallas import tpu_sc as plsc

BATCH = 16384
F = 512
LANES = 16
NC = 2
NS = 16
NW = NC * NS
ROWS_PER_W = BATCH // NW
R = 32
NCHUNK = ROWS_PER_W // R
NPAIR = NCHUNK // 2
NJ = F // LANES

_mesh = plsc.VectorSubcoreMesh(core_axis_name="c", subcore_axis_name="s")


@functools.partial(
    pl.kernel,
    out_type=jax.ShapeDtypeStruct((BATCH, F), jnp.float32),
    mesh=_mesh,
    compiler_params=pltpu.CompilerParams(needs_layout_passes=False),
    scratch_types=[
        pltpu.VMEM((F,), jnp.int32),
        pltpu.VMEM((R, F), jnp.float32),
        pltpu.VMEM((R, F), jnp.float32),
        pltpu.VMEM((R, F), jnp.float32),
        pltpu.VMEM((R, F), jnp.float32),
        pltpu.SemaphoreType.DMA,
        pltpu.SemaphoreType.DMA,
        pltpu.SemaphoreType.DMA,
        pltpu.SemaphoreType.DMA,
    ],
)
def _sc_gather(x_hbm, idx_hbm, out_hbm, idx_v, xb0, xb1, ob0, ob1,
               si0, si1, so0, so1):
    wid = lax.axis_index("s") * NC + lax.axis_index("c")
    base = wid * ROWS_PER_W

    pltpu.sync_copy(idx_hbm, idx_v)
    cols = [idx_v[pl.ds(j * LANES, LANES)] for j in range(NJ)]

    def start_in(ci, buf, sem):
        pltpu.async_copy(x_hbm.at[pl.ds(base + ci * R, R)], buf, sem)

    def start_out(ci, buf, sem):
        pltpu.async_copy(buf, out_hbm.at[pl.ds(base + ci * R, R)], sem)

    def wait_in(buf, sem):
        pltpu.make_async_copy(x_hbm.at[pl.ds(base, R)], buf, sem).wait()

    def wait_out(buf, sem):
        pltpu.make_async_copy(buf, out_hbm.at[pl.ds(base, R)], sem).wait()

    def gather(xb, ob):
        @plsc.parallel_loop(0, R, unroll=1)
        def _row(r):
            row = jnp.full((LANES,), r, dtype=jnp.int32)
            for j in range(NJ):
                vals = plsc.load_gather(xb, [row, cols[j]])
                ob[r, pl.ds(j * LANES, LANES)] = vals

    start_in(0, xb0, si0)
    start_in(1, xb1, si1)

    def pair_body(g, carry):
        for b, (xb, ob, si, so) in enumerate(
            ((xb0, ob0, si0, so0), (xb1, ob1, si1, so1))):
            ci = 2 * g + b
            wait_in(xb, si)

            @pl.when(g > 0)
            def _():
                wait_out(ob, so)

            gather(xb, ob)
            start_out(ci, ob, so)

            @pl.when(g < NPAIR - 1)
            def _():
                start_in(ci + 2, xb, si)
        return carry

    lax.fori_loop(0, NPAIR, pair_body, 0)

    wait_out(ob0, so0)
    wait_out(ob1, so1)


def kernel(x, sparse_index):
    return _sc_gather(x, sparse_index.astype(jnp.int32))


# speedup vs baseline: 1.4802x; 1.0200x over previous
import functools

import jax
import jax.numpy as jnp
from jax import lax
from jax.experimental import pallas as pl
from jax.experimental.pallas import tpu as pltpu
from jax.experimental.pallas import tpu_sc as plsc

BATCH = 16384
F = 512
LANES = 16
NC = 2
NS = 16
NW = NC * NS
ROWS_PER_W = BATCH // NW
R = 32
NCHUNK = ROWS_PER_W // R
NJ = F // LANES

_mesh = plsc.VectorSubcoreMesh(core_axis_name="c", subcore_axis_name="s")


@functools.partial(
    pl.kernel,
    out_type=jax.ShapeDtypeStruct((BATCH, F), jnp.float32),
    mesh=_mesh,
    compiler_params=pltpu.CompilerParams(needs_layout_passes=False),
    scratch_types=[
        pltpu.VMEM((F,), jnp.int32),
        pltpu.VMEM((2 * R, F), jnp.float32),
        pltpu.VMEM((2 * R, F), jnp.float32),
        pltpu.SemaphoreType.DMA,
        pltpu.SemaphoreType.DMA,
        pltpu.SemaphoreType.DMA,
        pltpu.SemaphoreType.DMA,
    ],
)
def _sc_gather(x_hbm, idx_hbm, out_hbm, idx_v, xb, ob, si0, si1, so0, so1):
    wid = lax.axis_index("s") * NC + lax.axis_index("c")
    base = wid * ROWS_PER_W

    pltpu.sync_copy(idx_hbm, idx_v)
    cols = [idx_v[pl.ds(j * LANES, LANES)] for j in range(NJ)]

    def start_in(ci, par, sem):
        pltpu.async_copy(x_hbm.at[pl.ds(base + ci * R, R)],
                         xb.at[pl.ds(par * R, R)], sem)

    def start_out(ci, par, sem):
        pltpu.async_copy(ob.at[pl.ds(par * R, R)],
                         out_hbm.at[pl.ds(base + ci * R, R)], sem)

    def wait_in(sem):
        pltpu.make_async_copy(x_hbm.at[pl.ds(base, R)],
                              xb.at[pl.ds(0, R)], sem).wait()

    def wait_out(sem):
        pltpu.make_async_copy(ob.at[pl.ds(0, R)],
                              out_hbm.at[pl.ds(base, R)], sem).wait()

    start_in(0, 0, si0)
    start_in(1, 1, si1)

    def chunk_body(ci, carry):
        par = lax.rem(ci, 2)
        even = par == 0

        @pl.when(even)
        def _():
            wait_in(si0)

        @pl.when(~even)
        def _():
            wait_in(si1)

        @pl.when(ci >= 2)
        def _():
            @pl.when(even)
            def _():
                wait_out(so0)

            @pl.when(~even)
            def _():
                wait_out(so1)

        row0 = par * R

        @plsc.parallel_loop(0, R, unroll=1)
        def _row(r):
            row = jnp.full((LANES,), row0 + r, dtype=jnp.int32)
            for j in range(NJ):
                vals = plsc.load_gather(xb, [row, cols[j]])
                ob[row0 + r, pl.ds(j * LANES, LANES)] = vals

        @pl.when(even)
        def _():
            start_out(ci, 0, so0)

        @pl.when(~even)
        def _():
            start_out(ci, 1, so1)

        @pl.when(jnp.logical_and(even, ci + 2 < NCHUNK))
        def _():
            start_in(ci + 2, 0, si0)

        @pl.when(jnp.logical_and(~even, ci + 2 < NCHUNK))
        def _():
            start_in(ci + 2, 1, si1)

        return carry

    lax.fori_loop(0, NCHUNK, chunk_body, 0)

    wait_out(so0)
    wait_out(so1)


def kernel(x, sparse_index):
    return _sc_gather(x, sparse_index.astype(jnp.int32))
